# baseline (speedup 1.0000x reference)
import functools

import jax
import jax.numpy as jnp
from jax import lax
from jax.experimental import pallas as pl
from jax.experimental.pallas import tpu as pltpu
from jax.experimental.pallas import tpu_sc as plsc

N = 10000
D = 128
N_PAD = 10240
PAD_ROW = N_PAD - 1
K = 128

_info = plsc.get_sparse_core_info()
NC = _info.num_cores
NS = _info.num_subcores
NW = NC * NS
RPS = N_PAD // NS

_mesh = plsc.VectorSubcoreMesh(core_axis_name="c", subcore_axis_name="s")


def _make_deg(epw, n_chunks):
    NBI = 4
    assert n_chunks % NBI == 0 and n_chunks >= NBI

    @functools.partial(
        pl.kernel,
        mesh=_mesh,
        out_type=jax.ShapeDtypeStruct((NC * N_PAD,), jnp.float32),
        scratch_types=(
            [pltpu.VMEM((K,), jnp.int32) for _ in range(NBI)]
            + [
                pltpu.VMEM((K,), jnp.float32),
                pltpu.VMEM((K,), jnp.float32),
                pltpu.VMEM_SHARED((N_PAD,), jnp.float32),
                pltpu.SemaphoreType.DMA((NBI,)),
            ]
        ),
    )
    def deg_kernel(dst_hbm, out_hbm, *rest):
        idxb = rest[:NBI]
        ones_v, zero_v, acc_sh, sem_i = rest[NBI:]
        c = lax.axis_index("c")
        s = lax.axis_index("s")
        wid = c * NS + s

        def initbuf(i, carry):
            ones_v[pl.ds(i * 16, 16)] = jnp.ones((16,), jnp.float32)
            zero_v[pl.ds(i * 16, 16)] = jnp.zeros((16,), jnp.float32)
            return carry

        lax.fori_loop(0, K // 16, initbuf, 0)

        for j in range(RPS // K):
            off = pl.multiple_of(s * RPS + j * K, 8)
            pltpu.sync_copy(zero_v, acc_sh.at[pl.ds(off, K)])
        plsc.subcore_barrier()

        base = wid * epw

        def idx_src(chunk):
            off = pl.multiple_of(base + chunk * K, 8)
            return dst_hbm.at[pl.ds(off, K)]

        for t in range(NBI):
            pltpu.async_copy(idx_src(t), idxb[t], sem_i.at[t])

        def group(i, carry):
            for t in range(NBI):
                ch = i * NBI + t
                pltpu.make_async_copy(idx_src(ch), idxb[t], sem_i.at[t]).wait()
                pltpu.sync_copy(ones_v, acc_sh.at[idxb[t]], add=True)

                @pl.when(ch + NBI < n_chunks)
                def _():
                    pltpu.async_copy(idx_src(ch + NBI), idxb[t], sem_i.at[t])

            return carry

        lax.fori_loop(0, n_chunks // NBI, group, 0)
        plsc.subcore_barrier()

        src_off = pl.multiple_of(s * RPS, 8)
        dst_off = pl.multiple_of(c * N_PAD + s * RPS, 8)
        pltpu.sync_copy(acc_sh.at[pl.ds(src_off, RPS)],
                        out_hbm.at[pl.ds(dst_off, RPS)])

    return deg_kernel


def _make_agg(epw, n_chunks):
    NR, NI = 2, 8
    assert n_chunks % NI == 0 and n_chunks >= NI

    @functools.partial(
        pl.kernel,
        mesh=_mesh,
        out_type=jax.ShapeDtypeStruct((NC * N_PAD, D), jnp.float32),
        scratch_types=(
            [pltpu.VMEM((2, K), jnp.int32) for _ in range(NI)]
            + [pltpu.VMEM((K, D), jnp.float32) for _ in range(NR)]
            + [
                pltpu.VMEM_SHARED((N_PAD, D), jnp.float32),
                pltpu.SemaphoreType.DMA((NI,)),
                pltpu.SemaphoreType.DMA((NR,)),
                pltpu.SemaphoreType.DMA((NR,)),
            ]
        ),
    )
    def agg_kernel(g_hbm, ipk_hbm, out_hbm, *rest):
        idxb = rest[:NI]
        rows = rest[NI:NI + NR]
        acc_sh, sem_i, sem_g, sem_s = rest[NI + NR:]
        c = lax.axis_index("c")
        s = lax.axis_index("s")
        wid = c * NS + s

        def zrow(i, carry):
            rows[0][i // (D // 16), pl.ds((i % (D // 16)) * 16, 16)] = (
                jnp.zeros((16,), jnp.float32))
            return carry

        lax.fori_loop(0, K * D // 16, zrow, 0)
        for j in range(RPS // K):
            pltpu.sync_copy(rows[0], acc_sh.at[pl.ds(s * RPS + j * K, K)])
        plsc.subcore_barrier()

        base = wid * n_chunks

        def idx_load(chunk, slot):
            return pltpu.async_copy(ipk_hbm.at[base + chunk], idxb[slot],
                                    sem_i.at[slot])

        def gather(chunk, islot, rslot):
            return pltpu.async_copy(g_hbm.at[idxb[islot].at[0]], rows[rslot],
                                    sem_g.at[rslot])

        def scatter(islot, rslot):
            return pltpu.async_copy(rows[rslot], acc_sh.at[idxb[islot].at[1]],
                                    sem_s.at[rslot], add=True)

        for t in range(NI - 2):
            idx_load(t, t)
        pltpu.make_async_copy(ipk_hbm.at[base], idxb[0], sem_i.at[0]).wait()
        gather(0, 0, 0)

        def group(i, carry):
            for t in range(NI):
                ch = i * NI + t
                b, b1 = t % NR, (t + 1) % NR
                i1, i6, i7 = (t + 1) % NI, (t + 6) % NI, (t + 7) % NI
                pltpu.make_async_copy(g_hbm.at[idxb[t].at[0]], rows[b],
                                      sem_g.at[b]).wait()
                scatter(t, b)

                @pl.when(ch >= 1)
                def _():
                    pltpu.make_async_copy(rows[b1],
                                          acc_sh.at[idxb[i7].at[1]],
                                          sem_s.at[b1]).wait()

                @pl.when(ch + 6 < n_chunks)
                def _():
                    idx_load(ch + 6, i6)

                @pl.when(ch + 1 < n_chunks)
                def _():
                    pltpu.make_async_copy(ipk_hbm.at[base + ch + 1],
                                          idxb[i1], sem_i.at[i1]).wait()
                    gather(ch + 1, i1, b1)

            return carry

        lax.fori_loop(0, n_chunks // NI, group, 0)
        ch = n_chunks - 1
        pltpu.make_async_copy(rows[ch % NR], acc_sh.at[idxb[ch % NI].at[1]],
                              sem_s.at[ch % NR]).wait()
        plsc.subcore_barrier()

        pltpu.sync_copy(acc_sh.at[pl.ds(s * RPS, RPS)],
                        out_hbm.at[pl.ds(c * N_PAD + s * RPS, RPS)])

    return agg_kernel


_BR = 1024
_NBLK = N_PAD // _BR


def _tc_first(x_pad, W1, dT):
    def body(x_ref, w_ref, dT_ref, g_ref, dinv_ref):
        dsum = dT_ref[:, 0:1] + dT_ref[:, 1:2]
        dinv = lax.rsqrt(1.0 + dsum)
        g_ref[...] = dinv * jnp.dot(x_ref[...], w_ref[...],
                                    preferred_element_type=jnp.float32)
        dinv_ref[...] = dinv

    return pl.pallas_call(
        body,
        grid=(_NBLK,),
        in_specs=[
            pl.BlockSpec((_BR, D), lambda i: (i, 0)),
            pl.BlockSpec((D, D), lambda i: (0, 0)),
            pl.BlockSpec((_BR, 2), lambda i: (i, 0)),
        ],
        out_specs=[
            pl.BlockSpec((_BR, D), lambda i: (i, 0)),
            pl.BlockSpec((_BR, 1), lambda i: (i, 0)),
        ],
        out_shape=[
            jax.ShapeDtypeStruct((N_PAD, D), jnp.float32),
            jax.ShapeDtypeStruct((N_PAD, 1), jnp.float32),
        ],
    )(x_pad, W1, dT)


def _tc_mid(p, g, dinv, b, W):
    def body(p0_ref, p1_ref, g_ref, dinv_ref, b_ref, w_ref, out_ref):
        t = dinv_ref[...] * (p0_ref[...] + p1_ref[...] + g_ref[...]) + b_ref[...]
        h = jnp.maximum(t, 0.0)
        out_ref[...] = dinv_ref[...] * jnp.dot(h, w_ref[...],
                                               preferred_element_type=jnp.float32)

    return pl.pallas_call(
        body,
        grid=(_NBLK,),
        in_specs=[
            pl.BlockSpec((_BR, D), lambda i: (i, 0)),
            pl.BlockSpec((_BR, D), lambda i: (i + _NBLK, 0)),
            pl.BlockSpec((_BR, D), lambda i: (i, 0)),
            pl.BlockSpec((_BR, 1), lambda i: (i, 0)),
            pl.BlockSpec((1, D), lambda i: (0, 0)),
            pl.BlockSpec((D, D), lambda i: (0, 0)),
        ],
        out_specs=pl.BlockSpec((_BR, D), lambda i: (i, 0)),
        out_shape=jax.ShapeDtypeStruct((N_PAD, D), jnp.float32),
    )(p, p, g, dinv, b, W)


def _tc_last(p, g, dinv, b):
    def body(p0_ref, p1_ref, g_ref, dinv_ref, b_ref, out_ref):
        t = dinv_ref[...] * (p0_ref[...] + p1_ref[...] + g_ref[...]) + b_ref[...]
        out_ref[...] = jnp.maximum(t, 0.0)

    return pl.pallas_call(
        body,
        grid=(-(-N // _BR),),
        in_specs=[
            pl.BlockSpec((_BR, D), lambda i: (i, 0)),
            pl.BlockSpec((_BR, D), lambda i: (i + _NBLK, 0)),
            pl.BlockSpec((_BR, D), lambda i: (i, 0)),
            pl.BlockSpec((_BR, 1), lambda i: (i, 0)),
            pl.BlockSpec((1, D), lambda i: (0, 0)),
        ],
        out_specs=pl.BlockSpec((_BR, D), lambda i: (i, 0)),
        out_shape=jax.ShapeDtypeStruct((N, D), jnp.float32),
    )(p, p, g, dinv, b)


def kernel(x, edge_index, W1, b1, W2, b2, W3, b3):
    E = edge_index.shape[1]
    n_chunks = -(-E // (NW * K))
    n_chunks = -(-n_chunks // 8) * 8
    epw = n_chunks * K
    e_pad = epw * NW

    src = edge_index[0]
    dst = edge_index[1]
    if e_pad > E:
        fill = jnp.full((e_pad - E,), PAD_ROW, dtype=jnp.int32)
        src = jnp.concatenate([src, fill])
        dst = jnp.concatenate([dst, fill])

    ipk = jnp.stack([src.reshape(NW * n_chunks, K),
                     dst.reshape(NW * n_chunks, K)], axis=1)

    x_pad = jnp.zeros((N_PAD, D), jnp.float32).at[:N, :].set(x)

    deg_fn = _make_deg(epw, n_chunks)
    agg_fn = _make_agg(epw, n_chunks)

    d = deg_fn(dst)
    dT = jnp.stack([d[:N_PAD], d[N_PAD:]], axis=1)

    g1, dinv = _tc_first(x_pad, W1, dT)
    p1 = agg_fn(g1, ipk)
    g2 = _tc_mid(p1, g1, dinv, b1.reshape(1, D), W2)
    p2 = agg_fn(g2, ipk)
    g3 = _tc_mid(p2, g2, dinv, b2.reshape(1, D), W3)
    p3 = agg_fn(g3, ipk)
    return _tc_last(p3, g3, dinv, b3.reshape(1, D))


# speedup vs baseline: 1.1159x; 1.1159x over previous
import functools

import jax
import jax.numpy as jnp
from jax import lax
from jax.experimental import pallas as pl
from jax.experimental.pallas import tpu as pltpu
from jax.experimental.pallas import tpu_sc as plsc

N = 10000
D = 128
N_PAD = 10240
PAD_ROW = N_PAD - 1
KA = 64
KD = 128
FM = 4

_info = plsc.get_sparse_core_info()
NC = _info.num_cores
NS = _info.num_subcores
NW = NC * NS
RPS = N_PAD // NS

_mesh = plsc.VectorSubcoreMesh(core_axis_name="c", subcore_axis_name="s")


def _make_deg(epw, n_chunks):
    NBI = 4
    assert n_chunks % NBI == 0 and n_chunks >= NBI

    @functools.partial(
        pl.kernel,
        mesh=_mesh,
        out_type=jax.ShapeDtypeStruct((NC * N_PAD,), jnp.float32),
        scratch_types=(
            [pltpu.VMEM((KD,), jnp.int32) for _ in range(NBI)]
            + [
                pltpu.VMEM((KD,), jnp.float32),
                pltpu.VMEM((KD,), jnp.float32),
                pltpu.VMEM_SHARED((N_PAD,), jnp.float32),
                pltpu.SemaphoreType.DMA((NBI,)),
            ]
        ),
    )
    def deg_kernel(dst_hbm, out_hbm, *rest):
        idxb = rest[:NBI]
        ones_v, zero_v, acc_sh, sem_i = rest[NBI:]
        c = lax.axis_index("c")
        s = lax.axis_index("s")
        wid = c * NS + s

        def initbuf(i, carry):
            ones_v[pl.ds(i * 16, 16)] = jnp.ones((16,), jnp.float32)
            zero_v[pl.ds(i * 16, 16)] = jnp.zeros((16,), jnp.float32)
            return carry

        lax.fori_loop(0, KD // 16, initbuf, 0)

        for j in range(RPS // KD):
            off = pl.multiple_of(s * RPS + j * KD, 8)
            pltpu.sync_copy(zero_v, acc_sh.at[pl.ds(off, KD)])
        plsc.subcore_barrier()

        base = wid * epw

        def idx_src(chunk):
            off = pl.multiple_of(base + chunk * KD, 8)
            return dst_hbm.at[pl.ds(off, KD)]

        for t in range(NBI):
            pltpu.async_copy(idx_src(t), idxb[t], sem_i.at[t])

        def group(i, carry):
            for t in range(NBI):
                ch = i * NBI + t
                pltpu.make_async_copy(idx_src(ch), idxb[t], sem_i.at[t]).wait()
                pltpu.sync_copy(ones_v, acc_sh.at[idxb[t]], add=True)

                @pl.when(ch + NBI < n_chunks)
                def _():
                    pltpu.async_copy(idx_src(ch + NBI), idxb[t], sem_i.at[t])

            return carry

        lax.fori_loop(0, n_chunks // NBI, group, 0)
        plsc.subcore_barrier()

        src_off = pl.multiple_of(s * RPS, 8)
        dst_off = pl.multiple_of(c * N_PAD + s * RPS, 8)
        pltpu.sync_copy(acc_sh.at[pl.ds(src_off, RPS)],
                        out_hbm.at[pl.ds(dst_off, RPS)])

    return deg_kernel


def _make_agg(ncf, ncs):
    NR, NI = 4, 8
    assert ncf % NI == 0 and ncs % NI == 0 and min(ncf, ncs) >= NI

    @functools.partial(
        pl.kernel,
        mesh=_mesh,
        out_type=jax.ShapeDtypeStruct((NC * N_PAD, D), jnp.float32),
        scratch_types=(
            [pltpu.VMEM((2, KA), jnp.int32) for _ in range(NI)]
            + [pltpu.VMEM((KA, D), jnp.float32) for _ in range(NR)]
            + [
                pltpu.VMEM_SHARED((N_PAD, D), jnp.float32),
                pltpu.SemaphoreType.DMA((NI,)),
                pltpu.SemaphoreType.DMA((NR,)),
                pltpu.SemaphoreType.DMA((NR,)),
            ]
        ),
    )
    def agg_kernel(g_hbm, ipk_hbm, out_hbm, *rest):
        idxb = rest[:NI]
        rows = rest[NI:NI + NR]
        acc_sh, sem_i, sem_g, sem_s = rest[NI + NR:]
        c = lax.axis_index("c")
        s = lax.axis_index("s")

        def zrow(i, carry):
            rows[0][i // (D // 16), pl.ds((i % (D // 16)) * 16, 16)] = (
                jnp.zeros((16,), jnp.float32))
            return carry

        lax.fori_loop(0, KA * D // 16, zrow, 0)
        for j in range(RPS // KA):
            pltpu.sync_copy(rows[0], acc_sh.at[pl.ds(s * RPS + j * KA, KA)])
        plsc.subcore_barrier()

        n_c = jnp.where(c == 0, ncf, ncs)
        base = jnp.where(c == 0, s * ncf, NS * ncf + s * ncs)

        def idx_load(chunk, slot):
            return pltpu.async_copy(ipk_hbm.at[base + chunk], idxb[slot],
                                    sem_i.at[slot])

        def gather(islot, rslot):
            return pltpu.async_copy(g_hbm.at[idxb[islot].at[0]], rows[rslot],
                                    sem_g.at[rslot])

        def scatter(islot, rslot):
            return pltpu.async_copy(rows[rslot], acc_sh.at[idxb[islot].at[1]],
                                    sem_s.at[rslot], add=True)

        for t in range(NI - 2):
            idx_load(t, t)
        for t in range(2):
            pltpu.make_async_copy(ipk_hbm.at[base + t], idxb[t],
                                  sem_i.at[t]).wait()
            gather(t, t)

        def group(i, carry):
            for t in range(NI):
                ch = i * NI + t
                b, b2 = t % NR, (t + 2) % NR
                i2, i6 = (t + 2) % NI, (t + 6) % NI
                pltpu.make_async_copy(g_hbm.at[idxb[t].at[0]], rows[b],
                                      sem_g.at[b]).wait()
                scatter(t, b)

                @pl.when(ch >= 2)
                def _():
                    pltpu.make_async_copy(rows[b2],
                                          acc_sh.at[idxb[i6].at[1]],
                                          sem_s.at[b2]).wait()

                @pl.when(ch + 6 < n_c)
                def _():
                    idx_load(ch + 6, i6)

                @pl.when(ch + 2 < n_c)
                def _():
                    pltpu.make_async_copy(ipk_hbm.at[base + ch + 2],
                                          idxb[i2], sem_i.at[i2]).wait()
                    gather(i2, b2)

            return carry

        lax.fori_loop(0, n_c // NI, group, 0)
        for dd in (2, 1):
            pltpu.make_async_copy(rows[(NI - dd) % NR],
                                  acc_sh.at[idxb[NI - dd].at[1]],
                                  sem_s.at[(NI - dd) % NR]).wait()
        plsc.subcore_barrier()

        pltpu.sync_copy(acc_sh.at[pl.ds(s * RPS, RPS)],
                        out_hbm.at[pl.ds(c * N_PAD + s * RPS, RPS)])

    return agg_kernel


_BR = 1024
_NBLK = N_PAD // _BR


def _tc_first(x_pad, W1, dT):
    def body(x_ref, w_ref, dT_ref, g_ref, dinv_ref):
        dsum = dT_ref[:, 0:1] + dT_ref[:, 1:2]
        dinv = lax.rsqrt(1.0 + dsum)
        g_ref[...] = dinv * jnp.dot(x_ref[...], w_ref[...],
                                    preferred_element_type=jnp.float32)
        dinv_ref[...] = dinv

    return pl.pallas_call(
        body,
        grid=(_NBLK,),
        in_specs=[
            pl.BlockSpec((_BR, D), lambda i: (i, 0)),
            pl.BlockSpec((D, D), lambda i: (0, 0)),
            pl.BlockSpec((_BR, 2), lambda i: (i, 0)),
        ],
        out_specs=[
            pl.BlockSpec((_BR, D), lambda i: (i, 0)),
            pl.BlockSpec((_BR, 1), lambda i: (i, 0)),
        ],
        out_shape=[
            jax.ShapeDtypeStruct((N_PAD, D), jnp.float32),
            jax.ShapeDtypeStruct((N_PAD, 1), jnp.float32),
        ],
    )(x_pad, W1, dT)


def _tc_mid(p, g, dinv, b, W):
    def body(p0_ref, p1_ref, g_ref, dinv_ref, b_ref, w_ref, out_ref):
        t = dinv_ref[...] * (p0_ref[...] + p1_ref[...] + g_ref[...]) + b_ref[...]
        h = jnp.maximum(t, 0.0)
        out_ref[...] = dinv_ref[...] * jnp.dot(h, w_ref[...],
                                               preferred_element_type=jnp.float32)

    return pl.pallas_call(
        body,
        grid=(_NBLK,),
        in_specs=[
            pl.BlockSpec((_BR, D), lambda i: (i, 0)),
            pl.BlockSpec((_BR, D), lambda i: (i + _NBLK, 0)),
            pl.BlockSpec((_BR, D), lambda i: (i, 0)),
            pl.BlockSpec((_BR, 1), lambda i: (i, 0)),
            pl.BlockSpec((1, D), lambda i: (0, 0)),
            pl.BlockSpec((D, D), lambda i: (0, 0)),
        ],
        out_specs=pl.BlockSpec((_BR, D), lambda i: (i, 0)),
        out_shape=jax.ShapeDtypeStruct((N_PAD, D), jnp.float32),
    )(p, p, g, dinv, b, W)


def _tc_last(p, g, dinv, b):
    def body(p0_ref, p1_ref, g_ref, dinv_ref, b_ref, out_ref):
        t = dinv_ref[...] * (p0_ref[...] + p1_ref[...] + g_ref[...]) + b_ref[...]
        out_ref[...] = jnp.maximum(t, 0.0)

    return pl.pallas_call(
        body,
        grid=(-(-N // _BR),),
        in_specs=[
            pl.BlockSpec((_BR, D), lambda i: (i, 0)),
            pl.BlockSpec((_BR, D), lambda i: (i + _NBLK, 0)),
            pl.BlockSpec((_BR, D), lambda i: (i, 0)),
            pl.BlockSpec((_BR, 1), lambda i: (i, 0)),
            pl.BlockSpec((1, D), lambda i: (0, 0)),
        ],
        out_specs=pl.BlockSpec((_BR, D), lambda i: (i, 0)),
        out_shape=jax.ShapeDtypeStruct((N, D), jnp.float32),
    )(p, p, g, dinv, b)


def kernel(x, edge_index, W1, b1, W2, b2, W3, b3):
    E = edge_index.shape[1]
    ncs = max(8, -(-(-(-E // (NS * (FM + 1) * KA))) // 8) * 8)
    ncf = FM * ncs
    e_pad = NS * (ncf + ncs) * KA

    src = edge_index[0]
    dst = edge_index[1]
    if e_pad > E:
        fill = jnp.full((e_pad - E,), PAD_ROW, dtype=jnp.int32)
        src = jnp.concatenate([src, fill])
        dst = jnp.concatenate([dst, fill])

    n_chunks_total = e_pad // KA
    ipk = jnp.stack([src.reshape(n_chunks_total, KA),
                     dst.reshape(n_chunks_total, KA)], axis=1)

    x_pad = jnp.zeros((N_PAD, D), jnp.float32).at[:N, :].set(x)

    epw_deg = e_pad // NW
    ndeg = epw_deg // KD
    deg_fn = _make_deg(epw_deg, ndeg)
    agg_fn = _make_agg(ncf, ncs)

    d = deg_fn(dst)
    dT = jnp.stack([d[:N_PAD], d[N_PAD:]], axis=1)

    g1, dinv = _tc_first(x_pad, W1, dT)
    p1 = agg_fn(g1, ipk)
    g2 = _tc_mid(p1, g1, dinv, b1.reshape(1, D), W2)
    p2 = agg_fn(g2, ipk)
    g3 = _tc_mid(p2, g2, dinv, b2.reshape(1, D), W3)
    p3 = agg_fn(g3, ipk)
    return _tc_last(p3, g3, dinv, b3.reshape(1, D))


# speedup vs baseline: 1.2110x; 1.0852x over previous
import functools

import jax
import jax.numpy as jnp
from jax import lax
from jax.experimental import pallas as pl
from jax.experimental.pallas import tpu as pltpu
from jax.experimental.pallas import tpu_sc as plsc

N = 10000
D = 128
N_PAD = 10240
PAD_ROW = N_PAD - 1
KA = 64
KD = 128

_info = plsc.get_sparse_core_info()
NC = _info.num_cores
NS = _info.num_subcores
NW = NC * NS
RPS = N_PAD // NS

_mesh = plsc.VectorSubcoreMesh(core_axis_name="c", subcore_axis_name="s")


def _make_deg(epw, n_chunks):
    NBI = 4
    assert n_chunks % NBI == 0 and n_chunks >= NBI

    @functools.partial(
        pl.kernel,
        mesh=_mesh,
        out_type=jax.ShapeDtypeStruct((NC * N_PAD,), jnp.float32),
        scratch_types=(
            [pltpu.VMEM((KD,), jnp.int32) for _ in range(NBI)]
            + [
                pltpu.VMEM((KD,), jnp.float32),
                pltpu.VMEM((KD,), jnp.float32),
                pltpu.VMEM_SHARED((N_PAD,), jnp.float32),
                pltpu.SemaphoreType.DMA((NBI,)),
            ]
        ),
    )
    def deg_kernel(dst_hbm, out_hbm, *rest):
        idxb = rest[:NBI]
        ones_v, zero_v, acc_sh, sem_i = rest[NBI:]
        c = lax.axis_index("c")
        s = lax.axis_index("s")
        wid = c * NS + s

        def initbuf(i, carry):
            ones_v[pl.ds(i * 16, 16)] = jnp.ones((16,), jnp.float32)
            zero_v[pl.ds(i * 16, 16)] = jnp.zeros((16,), jnp.float32)
            return carry

        lax.fori_loop(0, KD // 16, initbuf, 0)

        for j in range(RPS // KD):
            off = pl.multiple_of(s * RPS + j * KD, 8)
            pltpu.sync_copy(zero_v, acc_sh.at[pl.ds(off, KD)])
        plsc.subcore_barrier()

        base = wid * epw

        def idx_src(chunk):
            off = pl.multiple_of(base + chunk * KD, 8)
            return dst_hbm.at[pl.ds(off, KD)]

        for t in range(NBI):
            pltpu.async_copy(idx_src(t), idxb[t], sem_i.at[t])

        def group(i, carry):
            for t in range(NBI):
                ch = i * NBI + t
                pltpu.make_async_copy(idx_src(ch), idxb[t], sem_i.at[t]).wait()
                pltpu.sync_copy(ones_v, acc_sh.at[idxb[t]], add=True)

                @pl.when(ch + NBI < n_chunks)
                def _():
                    pltpu.async_copy(idx_src(ch + NBI), idxb[t], sem_i.at[t])

            return carry

        lax.fori_loop(0, n_chunks // NBI, group, 0)
        plsc.subcore_barrier()

        src_off = pl.multiple_of(s * RPS, 8)
        dst_off = pl.multiple_of(c * N_PAD + s * RPS, 8)
        pltpu.sync_copy(acc_sh.at[pl.ds(src_off, RPS)],
                        out_hbm.at[pl.ds(dst_off, RPS)])

    return deg_kernel


def _make_agg(ncf, ncs):
    NR, NI = 4, 8
    assert ncf % NI == 0 and ncs % NI == 0 and min(ncf, ncs) >= NI

    @functools.partial(
        pl.kernel,
        mesh=_mesh,
        out_type=jax.ShapeDtypeStruct((NC * N_PAD, D), jnp.float32),
        scratch_types=(
            [pltpu.VMEM((2, KA), jnp.int32) for _ in range(NI)]
            + [pltpu.VMEM((KA, D), jnp.float32) for _ in range(NR)]
            + [
                pltpu.VMEM_SHARED((N_PAD, D), jnp.float32),
                pltpu.SemaphoreType.DMA((NI,)),
                pltpu.SemaphoreType.DMA((NR,)),
                pltpu.SemaphoreType.DMA((NR,)),
            ]
        ),
    )
    def agg_kernel(g_hbm, ipk_hbm, out_hbm, *rest):
        idxb = rest[:NI]
        rows = rest[NI:NI + NR]
        acc_sh, sem_i, sem_g, sem_s = rest[NI + NR:]
        c = lax.axis_index("c")
        s = lax.axis_index("s")

        def zrow(i, carry):
            rows[0][i // (D // 16), pl.ds((i % (D // 16)) * 16, 16)] = (
                jnp.zeros((16,), jnp.float32))
            return carry

        lax.fori_loop(0, KA * D // 16, zrow, 0)
        for j in range(RPS // KA):
            pltpu.sync_copy(rows[0], acc_sh.at[pl.ds(s * RPS + j * KA, KA)])
        plsc.subcore_barrier()

        n_c = jnp.where(c == 0, ncf, ncs)
        base = jnp.where(c == 0, s * ncf, NS * ncf + s * ncs)

        def idx_load(chunk, slot):
            return pltpu.async_copy(ipk_hbm.at[base + chunk], idxb[slot],
                                    sem_i.at[slot])

        def gather(islot, rslot):
            return pltpu.async_copy(g_hbm.at[idxb[islot].at[0]], rows[rslot],
                                    sem_g.at[rslot])

        def scatter(islot, rslot):
            return pltpu.async_copy(rows[rslot], acc_sh.at[idxb[islot].at[1]],
                                    sem_s.at[rslot], add=True)

        for t in range(NI - 2):
            idx_load(t, t)
        for t in range(2):
            pltpu.make_async_copy(ipk_hbm.at[base + t], idxb[t],
                                  sem_i.at[t]).wait()
            gather(t, t)

        def group(i, carry):
            for t in range(NI):
                ch = i * NI + t
                b, b2 = t % NR, (t + 2) % NR
                i2, i6 = (t + 2) % NI, (t + 6) % NI
                pltpu.make_async_copy(g_hbm.at[idxb[t].at[0]], rows[b],
                                      sem_g.at[b]).wait()
                scatter(t, b)

                @pl.when(ch >= 2)
                def _():
                    pltpu.make_async_copy(rows[b2],
                                          acc_sh.at[idxb[i6].at[1]],
                                          sem_s.at[b2]).wait()

                @pl.when(ch + 6 < n_c)
                def _():
                    idx_load(ch + 6, i6)

                @pl.when(ch + 2 < n_c)
                def _():
                    pltpu.make_async_copy(ipk_hbm.at[base + ch + 2],
                                          idxb[i2], sem_i.at[i2]).wait()
                    gather(i2, b2)

            return carry

        lax.fori_loop(0, n_c // NI, group, 0)
        for dd in (2, 1):
            pltpu.make_async_copy(rows[(NI - dd) % NR],
                                  acc_sh.at[idxb[NI - dd].at[1]],
                                  sem_s.at[(NI - dd) % NR]).wait()
        plsc.subcore_barrier()

        pltpu.sync_copy(acc_sh.at[pl.ds(s * RPS, RPS)],
                        out_hbm.at[pl.ds(c * N_PAD + s * RPS, RPS)])

    return agg_kernel


_BR = 1024
_NBLK = N_PAD // _BR


def _tc_first(x_pad, W1, dT):
    def body(x_ref, w_ref, dT_ref, g_ref, dinv_ref):
        dsum = dT_ref[:, 0:1] + dT_ref[:, 1:2]
        dinv = lax.rsqrt(1.0 + dsum)
        g_ref[...] = dinv * jnp.dot(x_ref[...], w_ref[...],
                                    preferred_element_type=jnp.float32)
        dinv_ref[...] = dinv

    return pl.pallas_call(
        body,
        grid=(_NBLK,),
        in_specs=[
            pl.BlockSpec((_BR, D), lambda i: (i, 0)),
            pl.BlockSpec((D, D), lambda i: (0, 0)),
            pl.BlockSpec((_BR, 2), lambda i: (i, 0)),
        ],
        out_specs=[
            pl.BlockSpec((_BR, D), lambda i: (i, 0)),
            pl.BlockSpec((_BR, 1), lambda i: (i, 0)),
        ],
        out_shape=[
            jax.ShapeDtypeStruct((N_PAD, D), jnp.float32),
            jax.ShapeDtypeStruct((N_PAD, 1), jnp.float32),
        ],
    )(x_pad, W1, dT)


def _tc_mid(p, g, dinv, b, W):
    def body(p0_ref, p1_ref, g_ref, dinv_ref, b_ref, w_ref, out_ref):
        t = dinv_ref[...] * (p0_ref[...] + p1_ref[...] + g_ref[...]) + b_ref[...]
        h = jnp.maximum(t, 0.0)
        out_ref[...] = dinv_ref[...] * jnp.dot(h, w_ref[...],
                                               preferred_element_type=jnp.float32)

    return pl.pallas_call(
        body,
        grid=(_NBLK,),
        in_specs=[
            pl.BlockSpec((_BR, D), lambda i: (i, 0)),
            pl.BlockSpec((_BR, D), lambda i: (i + _NBLK, 0)),
            pl.BlockSpec((_BR, D), lambda i: (i, 0)),
            pl.BlockSpec((_BR, 1), lambda i: (i, 0)),
            pl.BlockSpec((1, D), lambda i: (0, 0)),
            pl.BlockSpec((D, D), lambda i: (0, 0)),
        ],
        out_specs=pl.BlockSpec((_BR, D), lambda i: (i, 0)),
        out_shape=jax.ShapeDtypeStruct((N_PAD, D), jnp.float32),
    )(p, p, g, dinv, b, W)


def _tc_last(p, g, dinv, b):
    def body(p0_ref, p1_ref, g_ref, dinv_ref, b_ref, out_ref):
        t = dinv_ref[...] * (p0_ref[...] + p1_ref[...] + g_ref[...]) + b_ref[...]
        out_ref[...] = jnp.maximum(t, 0.0)

    return pl.pallas_call(
        body,
        grid=(-(-N // _BR),),
        in_specs=[
            pl.BlockSpec((_BR, D), lambda i: (i, 0)),
            pl.BlockSpec((_BR, D), lambda i: (i + _NBLK, 0)),
            pl.BlockSpec((_BR, D), lambda i: (i, 0)),
            pl.BlockSpec((_BR, 1), lambda i: (i, 0)),
            pl.BlockSpec((1, D), lambda i: (0, 0)),
        ],
        out_specs=pl.BlockSpec((_BR, D), lambda i: (i, 0)),
        out_shape=jax.ShapeDtypeStruct((N, D), jnp.float32),
    )(p, p, g, dinv, b)


def kernel(x, edge_index, W1, b1, W2, b2, W3, b3):
    E = edge_index.shape[1]
    per_pair = -(--(-E // (NS * KA)) // 16) * 16
    ncs = max(8, (per_pair // 10) // 8 * 8)
    ncf = per_pair - ncs
    e_pad = NS * per_pair * KA

    src = edge_index[0]
    dst = edge_index[1]
    if e_pad > E:
        fill = jnp.full((e_pad - E,), PAD_ROW, dtype=jnp.int32)
        src = jnp.concatenate([src, fill])
        dst = jnp.concatenate([dst, fill])

    n_chunks_total = e_pad // KA
    ipk = jnp.stack([src.reshape(n_chunks_total, KA),
                     dst.reshape(n_chunks_total, KA)], axis=1)

    x_pad = jnp.zeros((N_PAD, D), jnp.float32).at[:N, :].set(x)

    epw_deg = e_pad // NW
    ndeg = epw_deg // KD
    deg_fn = _make_deg(epw_deg, ndeg)
    agg_fn = _make_agg(ncf, ncs)

    d = deg_fn(dst)
    dT = jnp.stack([d[:N_PAD], d[N_PAD:]], axis=1)

    g1, dinv = _tc_first(x_pad, W1, dT)
    p1 = agg_fn(g1, ipk)
    g2 = _tc_mid(p1, g1, dinv, b1.reshape(1, D), W2)
    p2 = agg_fn(g2, ipk)
    g3 = _tc_mid(p2, g2, dinv, b2.reshape(1, D), W3)
    p3 = agg_fn(g3, ipk)
    return _tc_last(p3, g3, dinv, b3.reshape(1, D))
